# Initial kernel scaffold; baseline (speedup 1.0000x reference)
import jax
import jax.numpy as jnp
from jax.experimental import pallas as pl


def kernel(category_indices, embedding_weight):
    raise NotImplementedError("write your pallas kernel here")


# speedup vs baseline: 1.0272x; 1.0272x over previous
import functools

import jax
import jax.numpy as jnp
from jax import lax
from jax.experimental import pallas as pl
from jax.experimental.pallas import tpu as pltpu
from jax.experimental.pallas import tpu_sc as plsc

NUM_CATEGORIES = 1000000
EMBEDDING_DIM = 64

NC = 2
NS = 16
NW = NC * NS

B_ROWS = 4096
B_COLS = 100
TOTAL = B_ROWS * B_COLS
PER_W = TOTAL // NW
GRP = 128
CHUNK_G = 10
CHUNK = GRP * CHUNK_G
NCHUNK = PER_W // CHUNK


def _sc_gather(idx_flat, table):
    mesh = plsc.VectorSubcoreMesh(
        core_axis_name="c", subcore_axis_name="s", num_cores=NC, num_subcores=NS
    )

    @functools.partial(
        pl.kernel,
        out_type=jax.ShapeDtypeStruct((TOTAL, EMBEDDING_DIM), jnp.float32),
        mesh=mesh,
        scratch_types=[
            pltpu.VMEM((CHUNK,), jnp.int32),
            pltpu.VMEM((CHUNK, EMBEDDING_DIM), jnp.float32),
            pltpu.SemaphoreType.DMA,
        ],
        compiler_params=pltpu.CompilerParams(use_tc_tiling_on_sc=False),
    )
    def k(idx_hbm, table_hbm, out_hbm, idx_v, rows_v, sem):
        wid = lax.axis_index("s") * NC + lax.axis_index("c")

        def chunk_body(c):
            base = wid * PER_W + c * CHUNK
            pltpu.sync_copy(idx_hbm.at[pl.ds(base, CHUNK)], idx_v)
            copies = []
            for g in range(CHUNK_G):
                cp = pltpu.async_copy(
                    table_hbm.at[idx_v.at[pl.ds(g * GRP, GRP)]],
                    rows_v.at[pl.ds(g * GRP, GRP)],
                    sem,
                )
                copies.append(cp)
            for cp in copies:
                cp.wait()
            obase = wid * PER_W + c * CHUNK
            pltpu.sync_copy(rows_v, out_hbm.at[pl.ds(obase, CHUNK)])

        pl.loop(0, NCHUNK)(chunk_body)

    return k(idx_flat, table)


def kernel(category_indices, embedding_weight):
    idx_flat = category_indices.astype(jnp.int32).reshape(TOTAL)
    flat = _sc_gather(idx_flat, embedding_weight)
    return flat.reshape(B_ROWS, B_COLS, EMBEDDING_DIM)
